# baseline (speedup 1.0000x reference)
import functools

import jax
import jax.numpy as jnp
from jax import lax
from jax.experimental import pallas as pl
from jax.experimental.pallas import tpu as pltpu
from jax.experimental.pallas import tpu_sc as plsc

_LN_EPS = 1e-5
_CH = 64
_NC = 2
_NS = 16
_NW = _NC * _NS


def _dense_body(xp_ref, xn_ref, g_ref, bt_ref, wr_ref, wn_ref, b_ref,
                root_ref, y_ref):
    g = g_ref[...]
    bt = bt_ref[...]

    def ln(v):
        mu = jnp.mean(v, axis=-1, keepdims=True)
        var = jnp.mean((v - mu) * (v - mu), axis=-1, keepdims=True)
        return (v - mu) * lax.rsqrt(var + _LN_EPS) * g + bt

    x = jnp.concatenate([ln(xp_ref[...]), ln(xn_ref[...])], axis=1)
    root_ref[...] = (
        jnp.dot(x, wr_ref[...], preferred_element_type=jnp.float32) + b_ref[...]
    )
    y_ref[...] = jnp.dot(x, wn_ref[...], preferred_element_type=jnp.float32)


def _combine_body(root_ref, p0_ref, p1_ref, out_ref):
    out_ref[...] = root_ref[...] + p0_ref[0] + p1_ref[0]


def _make_sc_kernel(n_pad, k, d):
    rows_per_sub = n_pad // _NS
    mesh = plsc.VectorSubcoreMesh(core_axis_name="c", subcore_axis_name="s")
    nbuf = 2
    nl = 16

    @functools.partial(
        pl.kernel,
        out_type=jax.ShapeDtypeStruct((_NC, n_pad, d), jnp.float32),
        mesh=mesh,
        scratch_types=[
            pltpu.VMEM((k, _CH), jnp.int32),
            pltpu.VMEM((3, _CH), jnp.int32),
            pltpu.VMEM((3, _CH), jnp.int32),
            pltpu.VMEM((nbuf, _CH, d), jnp.float32),
            pltpu.VMEM_SHARED((n_pad, d), jnp.float32),
            pltpu.SemaphoreType.DMA,
            pltpu.SemaphoreType.DMA,
        ],
    )
    def sc_kernel(y_hbm, ei_hbm, zeros_hbm, out_hbm,
                  pk_v, src_c, dst_c, rows_v, acc, gsem, ssem):
        c = lax.axis_index("c")
        s = lax.axis_index("s")
        w = c * _NS + s
        pltpu.sync_copy(ei_hbm.at[w], pk_v)
        row0 = s * rows_per_sub
        pltpu.sync_copy(zeros_hbm.at[pl.ds(row0, rows_per_sub)],
                        acc.at[pl.ds(row0, rows_per_sub)])
        plsc.subcore_barrier()

        def unpack(j, p):
            for i in range(_CH // nl):
                v = pk_v[j, pl.ds(i * nl, nl)]
                src_c[p, pl.ds(i * nl, nl)] = lax.bitwise_and(v, 0xFFFF)
                dst_c[p, pl.ds(i * nl, nl)] = lax.shift_right_logical(v, 16)

        def gather(j, p, slot):
            pltpu.async_copy(y_hbm.at[src_c.at[p]], rows_v.at[slot], gsem)

        def scatter(j, p, slot):
            pltpu.async_copy(rows_v.at[slot], acc.at[dst_c.at[p]], ssem,
                             add=True)

        def wait(sem):
            pltpu.make_async_copy(zeros_hbm.at[pl.ds(0, _CH)],
                                  rows_v.at[0], sem).wait()

        unpack(0, 0)
        gather(0, 0, 0)

        def body(j, carry):
            @pl.when(j >= 1)
            def _():
                wait(ssem)
            @pl.when(j + 1 < k)
            def _():
                jn = j + 1
                p = lax.rem(jn, 3)
                unpack(jn, p)
                gather(jn, p, lax.rem(jn, nbuf))
            wait(gsem)
            scatter(j, lax.rem(j, 3), lax.rem(j, nbuf))
            return carry

        lax.fori_loop(0, k, body, 0)
        wait(ssem)
        plsc.subcore_barrier()
        pltpu.sync_copy(acc.at[pl.ds(row0, rows_per_sub)],
                        out_hbm.at[c, pl.ds(row0, rows_per_sub)])

    return sc_kernel


def kernel(x_prev, x_same, x_next, edge_index, ln_gamma, ln_beta,
           W_root, W_neigh, b):
    n, d_prev = x_prev.shape
    d_out = W_root.shape[1]
    e = edge_index.shape[1]

    k = -(-e // (_NW * _CH))
    e_pad = _NW * _CH * k
    ny = n + 16
    n_pad = -(-(n + 1) // (_NS * 8)) * (_NS * 8)

    bn = 2000
    grid = (n // bn,)
    root, y = pl.pallas_call(
        _dense_body,
        grid=grid,
        in_specs=[
            pl.BlockSpec((bn, d_prev), lambda i: (i, 0)),
            pl.BlockSpec((bn, d_prev), lambda i: (i, 0)),
            pl.BlockSpec((1, d_prev), lambda i: (0, 0)),
            pl.BlockSpec((1, d_prev), lambda i: (0, 0)),
            pl.BlockSpec(W_root.shape, lambda i: (0, 0)),
            pl.BlockSpec(W_neigh.shape, lambda i: (0, 0)),
            pl.BlockSpec((1, d_out), lambda i: (0, 0)),
        ],
        out_specs=[
            pl.BlockSpec((bn, d_out), lambda i: (i, 0)),
            pl.BlockSpec((bn, d_out), lambda i: (i, 0)),
        ],
        out_shape=[
            jax.ShapeDtypeStruct((n, d_out), jnp.float32),
            jax.ShapeDtypeStruct((ny, d_out), jnp.float32),
        ],
    )(x_prev, x_next, ln_gamma.reshape(1, -1), ln_beta.reshape(1, -1),
      W_root, W_neigh, b.reshape(1, -1))

    npad_e = e_pad - e
    pad_src = jnp.full((npad_e,), n, jnp.int32)
    pad_dst = n + jnp.arange(npad_e, dtype=jnp.int32) % (n_pad - n)
    src = jnp.concatenate([edge_index[0], pad_src])
    dst = jnp.concatenate([edge_index[1], pad_dst])
    ei = (src | (dst << 16)).reshape(_NW, k, _CH)
    zeros = jnp.zeros((n_pad, d_out), jnp.float32)

    partials = _make_sc_kernel(n_pad, k, d_out)(y, ei, zeros)

    out = pl.pallas_call(
        _combine_body,
        grid=grid,
        in_specs=[
            pl.BlockSpec((bn, d_out), lambda i: (i, 0)),
            pl.BlockSpec((1, bn, d_out), lambda i: (0, i, 0)),
            pl.BlockSpec((1, bn, d_out), lambda i: (1, i, 0)),
        ],
        out_specs=pl.BlockSpec((bn, d_out), lambda i: (i, 0)),
        out_shape=jax.ShapeDtypeStruct((n, d_out), jnp.float32),
    )(root, partials, partials)
    return out
